# baseline (speedup 1.0000x reference)
import numpy as np
import jax
import jax.numpy as jnp
from jax.experimental import pallas as pl
from jax.experimental.pallas import tpu as pltpu

NUM_P = 4096
US = 8
NR = NUM_P * US
DX = 64
DU = 16

TINY = np.float32(np.finfo(np.float32).tiny)
LO_N = np.float32(np.nextafter(np.float32(-1.0), np.float32(0.0)))
SQRT2 = np.float32(np.sqrt(2.0))


def _tf2x32(k1, k2, x0, x1):
    ks2 = k1 ^ k2 ^ jnp.uint32(0x1BD11BDA)

    def rot(v, r):
        return (v << jnp.uint32(r)) | (v >> jnp.uint32(32 - r))

    def rounds(a, b, rs):
        for r in rs:
            a = a + b
            b = rot(b, r)
            b = a ^ b
        return a, b

    RA = (13, 15, 26, 6)
    RB = (17, 29, 16, 24)
    a = x0 + k1
    b = x1 + k2
    a, b = rounds(a, b, RA)
    a = a + k2
    b = b + ks2 + jnp.uint32(1)
    a, b = rounds(a, b, RB)
    a = a + ks2
    b = b + k1 + jnp.uint32(2)
    a, b = rounds(a, b, RA)
    a = a + k1
    b = b + k2 + jnp.uint32(3)
    a, b = rounds(a, b, RB)
    a = a + k2
    b = b + ks2 + jnp.uint32(4)
    a, b = rounds(a, b, RA)
    a = a + ks2
    b = b + k1 + jnp.uint32(5)
    return a, b


def _bits(k1, k2, p):
    a, b = _tf2x32(k1, k2, jnp.zeros_like(p), p)
    return a ^ b


def _u01(bits, lo, hi):
    fb = (bits >> jnp.uint32(9)) | jnp.uint32(0x3F800000)
    f = jax.lax.bitcast_convert_type(fb, jnp.float32) - jnp.float32(1.0)
    return jax.lax.max(lo, f * (hi - lo) + lo)


def _gumbel(bits):
    return -jnp.log(-jnp.log(_u01(bits, TINY, jnp.float32(1.0))))


def _normal(bits):
    return SQRT2 * jax.lax.erf_inv(_u01(bits, LO_N, jnp.float32(1.0)))


def _tree8(a):
    b = a[:, :4] + a[:, 4:]
    c = b[:, :2] + b[:, 2:]
    return c[:, 0:1] + c[:, 1:2]


def _red64(t):
    acc = t[:, 0:8]
    for v in range(1, 8):
        acc = acc + t[:, 8 * v:8 * v + 8]
    return _tree8(acc)


def _red16(t):
    return _tree8(t[:, 0:8] + t[:, 8:16])


def _prep_kernel(keys_ref, p_ref, krep_ref, q_ref, rrep_ref, nu_ref, lwu_ref):
    k1 = keys_ref[0]
    k2 = keys_ref[1]
    P = p_ref[...]
    mean = jnp.dot(P, krep_ref[...], preferred_element_type=jnp.float32)
    r = jax.lax.broadcasted_iota(jnp.uint32, (NUM_P, 128), 0)
    l = jax.lax.broadcasted_iota(jnp.uint32, (NUM_P, 128), 1)
    eps = _normal(_bits(k1, k2, r * jnp.uint32(128) + l))
    nu = mean + jnp.float32(0.1) * eps
    nu_ref[...] = nu
    costx = _red64(P * P * q_ref[...])
    t = nu * nu * rrep_ref[...]
    cols = []
    for u in range(8):
        cu = _red16(t[:, 16 * u:16 * u + 16])
        cols.append(costx + cu)
    lwu_ref[...] = -jnp.concatenate(cols, axis=1)


def _lse_kernel(lwu_ref, logw_ref):
    lwu = lwu_ref[...]
    m = jnp.max(lwu)
    s = jnp.sum(jnp.exp(lwu - m))
    lse = jnp.log(jnp.abs(s)) + m
    logw_ref[...] = lwu - lse


_IB = 8
_UNROLL = 4


def _argmax_kernel(keys_ref, logw_ref, samp_ref, lwsel_ref):
    k1 = keys_ref[2]
    k2 = keys_ref[3]
    i0 = pl.program_id(0) * _IB
    s_iota = jax.lax.broadcasted_iota(jnp.uint32, (_IB, 128), 0)
    l_iota = jax.lax.broadcasted_iota(jnp.uint32, (_IB, 128), 1)
    pbase = (jnp.uint32(i0) + s_iota) * jnp.uint32(NR) + l_iota
    l_i32 = jax.lax.broadcasted_iota(jnp.int32, (_IB, 128), 1)

    neg_inf = jnp.full((_IB, 128), -jnp.inf, jnp.float32)
    zero_i = jnp.zeros((_IB, 128), jnp.int32)

    def body(c, carry):
        vb, jb, lb = carry
        for u in range(_UNROLL):
            row = c * _UNROLL + u
            j0 = row * 128
            lw = logw_ref[pl.ds(row, 1), :]
            lw = jnp.broadcast_to(lw, (_IB, 128))
            g = _gumbel(_bits(k1, k2, pbase + jnp.uint32(j0)))
            v = g + lw
            upd = v > vb
            vb = jnp.where(upd, v, vb)
            jb = jnp.where(upd, l_i32 + j0, jb)
            lb = jnp.where(upd, lw, lb)
        return vb, jb, lb

    vb, jb, lb = jax.lax.fori_loop(
        0, 256 // _UNROLL, body, (neg_inf, zero_i, neg_inf))

    vmax = jnp.max(vb, axis=1, keepdims=True)
    mask = vb == vmax
    jcand = jnp.where(mask, jb, jnp.int32(2147483647))
    jmin = jnp.min(jcand, axis=1, keepdims=True)
    sel = mask & (jb == jmin)
    lwin = jnp.min(jnp.where(sel, lb, jnp.inf), axis=1, keepdims=True)
    samp_ref[...] = jmin
    lwsel_ref[...] = lwin


def _gather_kernel(s_ref, p_ref, nu_ref, gx_ref, gu_ref):
    del s_ref
    gx_ref[...] = p_ref[...]
    gu_ref[...] = nu_ref[...]


def _dyn_kernel(keys_ref, gx_ref, gu_ref, at_ref, bt_ref, out_ref):
    k1 = keys_ref[4]
    k2 = keys_ref[5]
    x1 = jnp.dot(gx_ref[...], at_ref[...], preferred_element_type=jnp.float32)
    x2 = jnp.dot(gu_ref[...], bt_ref[...], preferred_element_type=jnp.float32)
    r = jax.lax.broadcasted_iota(jnp.uint32, (NUM_P, DX), 0)
    l = jax.lax.broadcasted_iota(jnp.uint32, (NUM_P, DX), 1)
    eps = _normal(_bits(k1, k2, r * jnp.uint32(DX) + l))
    out_ref[...] = x1 + x2 + jnp.float32(0.01) * eps


def kernel(particles, K, A, B, Q, R, iteration):
    key = jax.random.fold_in(jax.random.key(1234), iteration)
    ku, ks, kn = jax.random.split(key, 3)
    keys = jnp.concatenate([jax.random.key_data(ku),
                            jax.random.key_data(ks),
                            jax.random.key_data(kn)]).astype(jnp.uint32)

    krep = jnp.tile(K.T, (1, US))
    rrep = jnp.tile(R.reshape(1, DU), (1, US))

    nu4, lwu = pl.pallas_call(
        _prep_kernel,
        in_specs=[pl.BlockSpec(memory_space=pltpu.SMEM),
                  pl.BlockSpec((NUM_P, DX), lambda: (0, 0)),
                  pl.BlockSpec((DX, 128), lambda: (0, 0)),
                  pl.BlockSpec((1, DX), lambda: (0, 0)),
                  pl.BlockSpec((1, 128), lambda: (0, 0))],
        out_specs=[pl.BlockSpec((NUM_P, 128), lambda: (0, 0)),
                   pl.BlockSpec((NUM_P, US), lambda: (0, 0))],
        out_shape=[jax.ShapeDtypeStruct((NUM_P, 128), jnp.float32),
                   jax.ShapeDtypeStruct((NUM_P, US), jnp.float32)],
    )(keys, particles, krep, Q.reshape(1, DX), rrep)

    new_u = nu4.reshape(NR, DU)

    logw = pl.pallas_call(
        _lse_kernel,
        in_specs=[pl.BlockSpec((256, 128), lambda: (0, 0))],
        out_specs=pl.BlockSpec((256, 128), lambda: (0, 0)),
        out_shape=jax.ShapeDtypeStruct((256, 128), jnp.float32),
    )(lwu.reshape(256, 128))

    samp, lwsel = pl.pallas_call(
        _argmax_kernel,
        grid=(NUM_P // _IB,),
        in_specs=[pl.BlockSpec(memory_space=pltpu.SMEM),
                  pl.BlockSpec((256, 128), lambda i: (0, 0))],
        out_specs=[pl.BlockSpec((_IB, 1), lambda i: (i, 0)),
                   pl.BlockSpec((_IB, 1), lambda i: (i, 0))],
        out_shape=[jax.ShapeDtypeStruct((NUM_P, 1), jnp.int32),
                   jax.ShapeDtypeStruct((NUM_P, 1), jnp.float32)],
    )(keys, logw)

    samples = samp.reshape(NUM_P)

    gx3, gu3 = pl.pallas_call(
        _gather_kernel,
        grid_spec=pltpu.PrefetchScalarGridSpec(
            num_scalar_prefetch=1,
            grid=(NUM_P,),
            in_specs=[pl.BlockSpec((1, 1, DX), lambda i, s: (s[i] // US, 0, 0)),
                      pl.BlockSpec((1, 1, DU), lambda i, s: (s[i], 0, 0))],
            out_specs=[pl.BlockSpec((1, 1, DX), lambda i, s: (i, 0, 0)),
                       pl.BlockSpec((1, 1, DU), lambda i, s: (i, 0, 0))],
        ),
        out_shape=[jax.ShapeDtypeStruct((NUM_P, 1, DX), jnp.float32),
                   jax.ShapeDtypeStruct((NUM_P, 1, DU), jnp.float32)],
    )(samples, particles.reshape(NUM_P, 1, DX), new_u.reshape(NR, 1, DU))
    gx = gx3.reshape(NUM_P, DX)
    gu = gu3.reshape(NUM_P, DU)

    new_particles = pl.pallas_call(
        _dyn_kernel,
        in_specs=[pl.BlockSpec(memory_space=pltpu.SMEM),
                  pl.BlockSpec((NUM_P, DX), lambda: (0, 0)),
                  pl.BlockSpec((NUM_P, DU), lambda: (0, 0)),
                  pl.BlockSpec((DX, DX), lambda: (0, 0)),
                  pl.BlockSpec((DU, DX), lambda: (0, 0))],
        out_specs=pl.BlockSpec((NUM_P, DX), lambda: (0, 0)),
        out_shape=jax.ShapeDtypeStruct((NUM_P, DX), jnp.float32),
    )(keys, gx, gu, A.T, B.T)

    joint = jnp.concatenate([gx, gu], axis=1)
    log_weights = lwsel.reshape(NUM_P)
    return (new_particles, joint, log_weights)


# speedup vs baseline: 1.8875x; 1.8875x over previous
import functools

import numpy as np
import jax
import jax.numpy as jnp
from jax import lax
from jax.experimental import pallas as pl
from jax.experimental.pallas import tpu as pltpu
from jax.experimental.pallas import tpu_sc as plsc

NUM_P = 4096
US = 8
NR = NUM_P * US
DX = 64
DU = 16

TINY = np.float32(np.finfo(np.float32).tiny)
LO_N = np.float32(np.nextafter(np.float32(-1.0), np.float32(0.0)))
SQRT2 = np.float32(np.sqrt(2.0))


def _tf2x32(k1, k2, x0, x1):
    ks2 = k1 ^ k2 ^ jnp.uint32(0x1BD11BDA)

    def rot(v, r):
        return (v << jnp.uint32(r)) | (v >> jnp.uint32(32 - r))

    def rounds(a, b, rs):
        for r in rs:
            a = a + b
            b = rot(b, r)
            b = a ^ b
        return a, b

    RA = (13, 15, 26, 6)
    RB = (17, 29, 16, 24)
    a = x0 + k1
    b = x1 + k2
    a, b = rounds(a, b, RA)
    a = a + k2
    b = b + ks2 + jnp.uint32(1)
    a, b = rounds(a, b, RB)
    a = a + ks2
    b = b + k1 + jnp.uint32(2)
    a, b = rounds(a, b, RA)
    a = a + k1
    b = b + k2 + jnp.uint32(3)
    a, b = rounds(a, b, RB)
    a = a + k2
    b = b + ks2 + jnp.uint32(4)
    a, b = rounds(a, b, RA)
    a = a + ks2
    b = b + k1 + jnp.uint32(5)
    return a, b


def _bits(k1, k2, p):
    a, b = _tf2x32(k1, k2, jnp.zeros_like(p), p)
    return a ^ b


def _u01(bits, lo, hi):
    fb = (bits >> jnp.uint32(9)) | jnp.uint32(0x3F800000)
    f = jax.lax.bitcast_convert_type(fb, jnp.float32) - jnp.float32(1.0)
    return jax.lax.max(lo, f * (hi - lo) + lo)


def _gumbel(bits):
    return -jnp.log(-jnp.log(_u01(bits, TINY, jnp.float32(1.0))))


def _normal(bits):
    return SQRT2 * jax.lax.erf_inv(_u01(bits, LO_N, jnp.float32(1.0)))


def _tree8(a):
    b = a[:, :4] + a[:, 4:]
    c = b[:, :2] + b[:, 2:]
    return c[:, 0:1] + c[:, 1:2]


def _red64(t):
    acc = t[:, 0:8]
    for v in range(1, 8):
        acc = acc + t[:, 8 * v:8 * v + 8]
    return _tree8(acc)


def _red16(t):
    return _tree8(t[:, 0:8] + t[:, 8:16])


def _prep_kernel(keys_ref, p_ref, krep_ref, q_ref, rrep_ref, t_ref, lwu_ref):
    k1 = keys_ref[0]
    k2 = keys_ref[1]
    P = p_ref[...]
    mean = jnp.dot(P, krep_ref[...], preferred_element_type=jnp.float32)
    r = jax.lax.broadcasted_iota(jnp.uint32, (NUM_P, 128), 0)
    l = jax.lax.broadcasted_iota(jnp.uint32, (NUM_P, 128), 1)
    eps = _normal(_bits(k1, k2, r * jnp.uint32(128) + l))
    nu = mean + jnp.float32(0.1) * eps
    t_ref[:, 0:DX] = P
    t_ref[:, DX:DX + 128] = nu
    t_ref[:, DX + 128:] = jnp.zeros((NUM_P, 256 - DX - 128), jnp.float32)
    costx = _red64(P * P * q_ref[...])
    t = nu * nu * rrep_ref[...]
    cols = []
    for u in range(8):
        cu = _red16(t[:, 16 * u:16 * u + 16])
        cols.append(costx + cu)
    lwu_ref[...] = -jnp.concatenate(cols, axis=1)


def _lse_kernel(lwu_ref, logw_ref):
    lwu = lwu_ref[...]
    m = jnp.max(lwu)
    s = jnp.sum(jnp.exp(lwu - m))
    lse = jnp.log(jnp.abs(s)) + m
    logw_ref[...] = lwu - lse


_IB = 16
_UNROLL = 4


def _argmax_kernel(keys_ref, logw_ref, samp_ref, lwsel_ref):
    k1 = keys_ref[2]
    k2 = keys_ref[3]
    i0 = pl.program_id(0) * _IB
    s_iota = jax.lax.broadcasted_iota(jnp.uint32, (_IB, 128), 0)
    l_iota = jax.lax.broadcasted_iota(jnp.uint32, (_IB, 128), 1)
    pbase = (jnp.uint32(i0) + s_iota) * jnp.uint32(NR) + l_iota
    l_i32 = jax.lax.broadcasted_iota(jnp.int32, (_IB, 128), 1)

    neg_inf = jnp.full((_IB, 128), -jnp.inf, jnp.float32)
    zero_i = jnp.zeros((_IB, 128), jnp.int32)

    def body(c, carry):
        vb, jb, lb = carry
        for u in range(_UNROLL):
            row = c * _UNROLL + u
            j0 = row * 128
            lw = logw_ref[pl.ds(row, 1), :]
            lw = jnp.broadcast_to(lw, (_IB, 128))
            g = _gumbel(_bits(k1, k2, pbase + jnp.uint32(j0)))
            v = g + lw
            upd = v > vb
            vb = jnp.where(upd, v, vb)
            jb = jnp.where(upd, l_i32 + j0, jb)
            lb = jnp.where(upd, lw, lb)
        return vb, jb, lb

    vb, jb, lb = jax.lax.fori_loop(
        0, 256 // _UNROLL, body, (neg_inf, zero_i, neg_inf))

    vmax = jnp.max(vb, axis=1, keepdims=True)
    mask = vb == vmax
    jcand = jnp.where(mask, jb, jnp.int32(2147483647))
    jmin = jnp.min(jcand, axis=1, keepdims=True)
    sel = mask & (jb == jmin)
    lwin = jnp.min(jnp.where(sel, lb, jnp.inf), axis=1, keepdims=True)
    samp_ref[...] = jmin
    lwsel_ref[...] = lwin


_NW = 32
_BPW = NUM_P // _NW


def _sc_gather_body(idx_hbm, t_hbm, g_hbm,
                    idx_v, idx2_v, g_v, sem):
    wid = lax.axis_index("s") * 2 + lax.axis_index("c")
    base = wid * _BPW
    pltpu.sync_copy(idx_hbm.at[pl.ds(base, _BPW)], idx_v)
    for i in range(_BPW // 16):
        sl = pl.ds(i * 16, 16)
        idx2_v[sl] = lax.shift_right_arithmetic(idx_v[sl], 3)
    pltpu.async_copy(t_hbm.at[idx2_v], g_v, sem).wait()
    pltpu.sync_copy(g_v, g_hbm.at[pl.ds(base, _BPW)])


_sc_gather = functools.partial(
    pl.kernel, _sc_gather_body,
    mesh=plsc.VectorSubcoreMesh(core_axis_name="c", subcore_axis_name="s"),
    out_type=jax.ShapeDtypeStruct((NUM_P, 256), jnp.float32),
    scratch_types=[pltpu.VMEM((_BPW,), jnp.int32),
                   pltpu.VMEM((_BPW,), jnp.int32),
                   pltpu.VMEM((_BPW, 256), jnp.float32),
                   pltpu.SemaphoreType.DMA])


def _dyn_kernel(keys_ref, g_ref, samp_ref, at_ref, bt_ref, out_ref, us_ref):
    k1 = keys_ref[4]
    k2 = keys_ref[5]
    xs = g_ref[:, 0:DX]
    rep = lax.rem(samp_ref[...], jnp.int32(US))
    us = jnp.zeros((NUM_P, DU), jnp.float32)
    for u in range(US):
        sl = g_ref[:, DX + 16 * u:DX + 16 * u + 16]
        us = jnp.where(rep == u, sl, us)
    us_ref[...] = us
    x1 = jnp.dot(xs, at_ref[...], preferred_element_type=jnp.float32)
    x2 = jnp.dot(us, bt_ref[...], preferred_element_type=jnp.float32)
    r = jax.lax.broadcasted_iota(jnp.uint32, (NUM_P, DX), 0)
    l = jax.lax.broadcasted_iota(jnp.uint32, (NUM_P, DX), 1)
    eps = _normal(_bits(k1, k2, r * jnp.uint32(DX) + l))
    out_ref[...] = x1 + x2 + jnp.float32(0.01) * eps


def kernel(particles, K, A, B, Q, R, iteration):
    key = jax.random.fold_in(jax.random.key(1234), iteration)
    ku, ks, kn = jax.random.split(key, 3)
    keys = jnp.concatenate([jax.random.key_data(ku),
                            jax.random.key_data(ks),
                            jax.random.key_data(kn)]).astype(jnp.uint32)

    krep = jnp.tile(K.T, (1, US))
    rrep = jnp.tile(R.reshape(1, DU), (1, US))

    table, lwu = pl.pallas_call(
        _prep_kernel,
        in_specs=[pl.BlockSpec(memory_space=pltpu.SMEM),
                  pl.BlockSpec((NUM_P, DX), lambda: (0, 0)),
                  pl.BlockSpec((DX, 128), lambda: (0, 0)),
                  pl.BlockSpec((1, DX), lambda: (0, 0)),
                  pl.BlockSpec((1, 128), lambda: (0, 0))],
        out_specs=[pl.BlockSpec((NUM_P, 256), lambda: (0, 0)),
                   pl.BlockSpec((NUM_P, US), lambda: (0, 0))],
        out_shape=[jax.ShapeDtypeStruct((NUM_P, 256), jnp.float32),
                   jax.ShapeDtypeStruct((NUM_P, US), jnp.float32)],
    )(keys, particles, krep, Q.reshape(1, DX), rrep)

    logw = pl.pallas_call(
        _lse_kernel,
        in_specs=[pl.BlockSpec((256, 128), lambda: (0, 0))],
        out_specs=pl.BlockSpec((256, 128), lambda: (0, 0)),
        out_shape=jax.ShapeDtypeStruct((256, 128), jnp.float32),
    )(lwu.reshape(256, 128))

    samp, lwsel = pl.pallas_call(
        _argmax_kernel,
        grid=(NUM_P // _IB,),
        in_specs=[pl.BlockSpec(memory_space=pltpu.SMEM),
                  pl.BlockSpec((256, 128), lambda i: (0, 0))],
        out_specs=[pl.BlockSpec((_IB, 1), lambda i: (i, 0)),
                   pl.BlockSpec((_IB, 1), lambda i: (i, 0))],
        out_shape=[jax.ShapeDtypeStruct((NUM_P, 1), jnp.int32),
                   jax.ShapeDtypeStruct((NUM_P, 1), jnp.float32)],
    )(keys, logw)

    samples = samp.reshape(NUM_P)

    g = _sc_gather()(samples, table)

    new_particles, gu = pl.pallas_call(
        _dyn_kernel,
        in_specs=[pl.BlockSpec(memory_space=pltpu.SMEM),
                  pl.BlockSpec((NUM_P, 256), lambda: (0, 0)),
                  pl.BlockSpec((NUM_P, 1), lambda: (0, 0)),
                  pl.BlockSpec((DX, DX), lambda: (0, 0)),
                  pl.BlockSpec((DU, DX), lambda: (0, 0))],
        out_specs=[pl.BlockSpec((NUM_P, DX), lambda: (0, 0)),
                   pl.BlockSpec((NUM_P, DU), lambda: (0, 0))],
        out_shape=[jax.ShapeDtypeStruct((NUM_P, DX), jnp.float32),
                   jax.ShapeDtypeStruct((NUM_P, DU), jnp.float32)],
    )(keys, g, samp, A.T, B.T)

    joint = jnp.concatenate([g[:, 0:DX], gu], axis=1)
    log_weights = lwsel.reshape(NUM_P)
    return (new_particles, joint, log_weights)


# speedup vs baseline: 1.9600x; 1.0384x over previous
import functools

import numpy as np
import jax
import jax.numpy as jnp
from jax import lax
from jax.experimental import pallas as pl
from jax.experimental.pallas import tpu as pltpu
from jax.experimental.pallas import tpu_sc as plsc

NUM_P = 4096
US = 8
NR = NUM_P * US
DX = 64
DU = 16

TINY = np.float32(np.finfo(np.float32).tiny)
LO_N = np.float32(np.nextafter(np.float32(-1.0), np.float32(0.0)))
SQRT2 = np.float32(np.sqrt(2.0))


def _tf2x32(k1, k2, x0, x1):
    ks2 = k1 ^ k2 ^ jnp.uint32(0x1BD11BDA)

    def rot(v, r):
        return (v << jnp.uint32(r)) | (v >> jnp.uint32(32 - r))

    def rounds(a, b, rs):
        for r in rs:
            a = a + b
            b = rot(b, r)
            b = a ^ b
        return a, b

    RA = (13, 15, 26, 6)
    RB = (17, 29, 16, 24)
    a = x0 + k1
    b = x1 + k2
    a, b = rounds(a, b, RA)
    a = a + k2
    b = b + ks2 + jnp.uint32(1)
    a, b = rounds(a, b, RB)
    a = a + ks2
    b = b + k1 + jnp.uint32(2)
    a, b = rounds(a, b, RA)
    a = a + k1
    b = b + k2 + jnp.uint32(3)
    a, b = rounds(a, b, RB)
    a = a + k2
    b = b + ks2 + jnp.uint32(4)
    a, b = rounds(a, b, RA)
    a = a + ks2
    b = b + k1 + jnp.uint32(5)
    return a, b


def _bits(k1, k2, p):
    a, b = _tf2x32(k1, k2, jnp.zeros_like(p), p)
    return a ^ b


def _u01(bits, lo, hi):
    fb = (bits >> jnp.uint32(9)) | jnp.uint32(0x3F800000)
    f = jax.lax.bitcast_convert_type(fb, jnp.float32) - jnp.float32(1.0)
    return jax.lax.max(lo, f * (hi - lo) + lo)


def _gumbel(bits):
    return -jnp.log(-jnp.log(_u01(bits, TINY, jnp.float32(1.0))))


def _normal(bits):
    return SQRT2 * jax.lax.erf_inv(_u01(bits, LO_N, jnp.float32(1.0)))


def _tree8(a):
    b = a[:, :4] + a[:, 4:]
    c = b[:, :2] + b[:, 2:]
    return c[:, 0:1] + c[:, 1:2]


def _red64(t):
    acc = t[:, 0:8]
    for v in range(1, 8):
        acc = acc + t[:, 8 * v:8 * v + 8]
    return _tree8(acc)


def _red16(t):
    return _tree8(t[:, 0:8] + t[:, 8:16])


def _prep_kernel(keys_ref, p_ref, krep_ref, q_ref, rrep_ref, t_ref, lwu_ref):
    k1 = keys_ref[0]
    k2 = keys_ref[1]
    P = p_ref[...]
    mean = jnp.dot(P, krep_ref[...], preferred_element_type=jnp.float32)
    r = jax.lax.broadcasted_iota(jnp.uint32, (NUM_P, 128), 0)
    l = jax.lax.broadcasted_iota(jnp.uint32, (NUM_P, 128), 1)
    eps = _normal(_bits(k1, k2, r * jnp.uint32(128) + l))
    nu = mean + jnp.float32(0.1) * eps
    t_ref[:, 0:DX] = P
    t_ref[:, DX:DX + 128] = nu
    t_ref[:, DX + 128:] = jnp.zeros((NUM_P, 256 - DX - 128), jnp.float32)
    costx = _red64(P * P * q_ref[...])
    t = nu * nu * rrep_ref[...]
    cols = []
    for u in range(8):
        cu = _red16(t[:, 16 * u:16 * u + 16])
        cols.append(costx + cu)
    lwu_ref[...] = -jnp.concatenate(cols, axis=1)


def _lse_kernel(lwu_ref, logw_ref):
    lwu = lwu_ref[...]
    m = jnp.max(lwu)
    s = jnp.sum(jnp.exp(lwu - m))
    lse = jnp.log(jnp.abs(s)) + m
    logw_ref[...] = lwu - lse


_NW = 32

_IB = 32
_UNROLL = 2


def _argmax_kernel(keys_ref, logw_ref, samp_ref, lwsel_ref):
    k1 = keys_ref[2]
    k2 = keys_ref[3]
    i0 = pl.program_id(0) * _IB
    s_iota = jax.lax.broadcasted_iota(jnp.uint32, (_IB, 128), 0)
    l_iota = jax.lax.broadcasted_iota(jnp.uint32, (_IB, 128), 1)
    pbase = (jnp.uint32(i0) + s_iota) * jnp.uint32(NR) + l_iota
    l_i32 = jax.lax.broadcasted_iota(jnp.int32, (_IB, 128), 1)

    neg_inf = jnp.full((_IB, 128), -jnp.inf, jnp.float32)
    zero_i = jnp.zeros((_IB, 128), jnp.int32)

    def body(c, carry):
        vb, jb, lb = carry
        for u in range(_UNROLL):
            row = c * _UNROLL + u
            j0 = row * 128
            lw = logw_ref[pl.ds(row, 1), :]
            lw = jnp.broadcast_to(lw, (_IB, 128))
            g = _gumbel(_bits(k1, k2, pbase + jnp.uint32(j0)))
            v = g + lw
            upd = v > vb
            vb = jnp.where(upd, v, vb)
            jb = jnp.where(upd, l_i32 + j0, jb)
            lb = jnp.where(upd, lw, lb)
        return vb, jb, lb

    vb, jb, lb = jax.lax.fori_loop(
        0, 256 // _UNROLL, body, (neg_inf, zero_i, neg_inf))

    vmax = jnp.max(vb, axis=1, keepdims=True)
    mask = vb == vmax
    jcand = jnp.where(mask, jb, jnp.int32(2147483647))
    jmin = jnp.min(jcand, axis=1, keepdims=True)
    sel = mask & (jb == jmin)
    lwin = jnp.min(jnp.where(sel, lb, jnp.inf), axis=1, keepdims=True)
    samp_ref[...] = jmin
    lwsel_ref[...] = lwin


_BPW = NUM_P // _NW


def _sc_gather_body(idx_hbm, t_hbm, g_hbm,
                    idx_v, idx2_v, g_v, sem):
    wid = lax.axis_index("s") * 2 + lax.axis_index("c")
    base = wid * _BPW
    pltpu.sync_copy(idx_hbm.at[pl.ds(base, _BPW)], idx_v)
    for i in range(_BPW // 16):
        sl = pl.ds(i * 16, 16)
        idx2_v[sl] = lax.shift_right_arithmetic(idx_v[sl], 3)
    pltpu.async_copy(t_hbm.at[idx2_v], g_v, sem).wait()
    pltpu.sync_copy(g_v, g_hbm.at[pl.ds(base, _BPW)])


def _sc_gather():
    return pl.kernel(
        _sc_gather_body,
        mesh=plsc.VectorSubcoreMesh(core_axis_name="c", subcore_axis_name="s"),
        out_type=jax.ShapeDtypeStruct((NUM_P, 256), jnp.float32),
        scratch_types=[pltpu.VMEM((_BPW,), jnp.int32),
                       pltpu.VMEM((_BPW,), jnp.int32),
                       pltpu.VMEM((_BPW, 256), jnp.float32),
                       pltpu.SemaphoreType.DMA])


def _dyn_kernel(keys_ref, g_ref, samp_ref, at_ref, bt_ref, out_ref, us_ref):
    k1 = keys_ref[4]
    k2 = keys_ref[5]
    xs = g_ref[:, 0:DX]
    rep = lax.rem(samp_ref[...], jnp.int32(US))
    us = jnp.zeros((NUM_P, DU), jnp.float32)
    for u in range(US):
        sl = g_ref[:, DX + 16 * u:DX + 16 * u + 16]
        us = jnp.where(rep == u, sl, us)
    us_ref[...] = us
    x1 = jnp.dot(xs, at_ref[...], preferred_element_type=jnp.float32)
    x2 = jnp.dot(us, bt_ref[...], preferred_element_type=jnp.float32)
    r = jax.lax.broadcasted_iota(jnp.uint32, (NUM_P, DX), 0)
    l = jax.lax.broadcasted_iota(jnp.uint32, (NUM_P, DX), 1)
    eps = _normal(_bits(k1, k2, r * jnp.uint32(DX) + l))
    out_ref[...] = x1 + x2 + jnp.float32(0.01) * eps


def kernel(particles, K, A, B, Q, R, iteration):
    key = jax.random.fold_in(jax.random.key(1234), iteration)
    ku, ks, kn = jax.random.split(key, 3)
    keys = jnp.concatenate([jax.random.key_data(ku),
                            jax.random.key_data(ks),
                            jax.random.key_data(kn)]).astype(jnp.uint32)

    krep = jnp.tile(K.T, (1, US))
    rrep = jnp.tile(R.reshape(1, DU), (1, US))

    table, lwu = pl.pallas_call(
        _prep_kernel,
        in_specs=[pl.BlockSpec(memory_space=pltpu.SMEM),
                  pl.BlockSpec((NUM_P, DX), lambda: (0, 0)),
                  pl.BlockSpec((DX, 128), lambda: (0, 0)),
                  pl.BlockSpec((1, DX), lambda: (0, 0)),
                  pl.BlockSpec((1, 128), lambda: (0, 0))],
        out_specs=[pl.BlockSpec((NUM_P, 256), lambda: (0, 0)),
                   pl.BlockSpec((NUM_P, US), lambda: (0, 0))],
        out_shape=[jax.ShapeDtypeStruct((NUM_P, 256), jnp.float32),
                   jax.ShapeDtypeStruct((NUM_P, US), jnp.float32)],
    )(keys, particles, krep, Q.reshape(1, DX), rrep)

    logw = pl.pallas_call(
        _lse_kernel,
        in_specs=[pl.BlockSpec((256, 128), lambda: (0, 0))],
        out_specs=pl.BlockSpec((256, 128), lambda: (0, 0)),
        out_shape=jax.ShapeDtypeStruct((256, 128), jnp.float32),
    )(lwu.reshape(256, 128))

    samp, lwsel = pl.pallas_call(
        _argmax_kernel,
        grid=(NUM_P // _IB,),
        in_specs=[pl.BlockSpec(memory_space=pltpu.SMEM),
                  pl.BlockSpec((256, 128), lambda i: (0, 0))],
        out_specs=[pl.BlockSpec((_IB, 1), lambda i: (i, 0)),
                   pl.BlockSpec((_IB, 1), lambda i: (i, 0))],
        out_shape=[jax.ShapeDtypeStruct((NUM_P, 1), jnp.int32),
                   jax.ShapeDtypeStruct((NUM_P, 1), jnp.float32)],
    )(keys, logw)

    samples = samp.reshape(NUM_P)

    g = _sc_gather()(samples, table)

    new_particles, gu = pl.pallas_call(
        _dyn_kernel,
        in_specs=[pl.BlockSpec(memory_space=pltpu.SMEM),
                  pl.BlockSpec((NUM_P, 256), lambda: (0, 0)),
                  pl.BlockSpec((NUM_P, 1), lambda: (0, 0)),
                  pl.BlockSpec((DX, DX), lambda: (0, 0)),
                  pl.BlockSpec((DU, DX), lambda: (0, 0))],
        out_specs=[pl.BlockSpec((NUM_P, DX), lambda: (0, 0)),
                   pl.BlockSpec((NUM_P, DU), lambda: (0, 0))],
        out_shape=[jax.ShapeDtypeStruct((NUM_P, DX), jnp.float32),
                   jax.ShapeDtypeStruct((NUM_P, DU), jnp.float32)],
    )(keys, g, samp, A.T, B.T)

    joint = jnp.concatenate([g[:, 0:DX], gu], axis=1)
    log_weights = lwsel.reshape(NUM_P)
    return (new_particles, joint, log_weights)


# speedup vs baseline: 2.1126x; 1.0779x over previous
import functools

import numpy as np
import jax
import jax.numpy as jnp
from jax import lax
from jax.experimental import pallas as pl
from jax.experimental.pallas import tpu as pltpu
from jax.experimental.pallas import tpu_sc as plsc

NUM_P = 4096
US = 8
NR = NUM_P * US
DX = 64
DU = 16

TINY = np.float32(np.finfo(np.float32).tiny)
LO_N = np.float32(np.nextafter(np.float32(-1.0), np.float32(0.0)))
SQRT2 = np.float32(np.sqrt(2.0))


def _tf2x32(k1, k2, x0, x1):
    ks2 = k1 ^ k2 ^ jnp.uint32(0x1BD11BDA)

    def rot(v, r):
        return (v << jnp.uint32(r)) | (v >> jnp.uint32(32 - r))

    def rounds(a, b, rs):
        for r in rs:
            a = a + b
            b = rot(b, r)
            b = a ^ b
        return a, b

    RA = (13, 15, 26, 6)
    RB = (17, 29, 16, 24)
    a = x0 + k1
    b = x1 + k2
    a, b = rounds(a, b, RA)
    a = a + k2
    b = b + ks2 + jnp.uint32(1)
    a, b = rounds(a, b, RB)
    a = a + ks2
    b = b + k1 + jnp.uint32(2)
    a, b = rounds(a, b, RA)
    a = a + k1
    b = b + k2 + jnp.uint32(3)
    a, b = rounds(a, b, RB)
    a = a + k2
    b = b + ks2 + jnp.uint32(4)
    a, b = rounds(a, b, RA)
    a = a + ks2
    b = b + k1 + jnp.uint32(5)
    return a, b


def _bits(k1, k2, p):
    a, b = _tf2x32(k1, k2, jnp.zeros_like(p), p)
    return a ^ b


def _u01(bits, lo, hi):
    fb = (bits >> jnp.uint32(9)) | jnp.uint32(0x3F800000)
    f = jax.lax.bitcast_convert_type(fb, jnp.float32) - jnp.float32(1.0)
    return jax.lax.max(lo, f * (hi - lo) + lo)


def _gumbel(bits):
    return -jnp.log(-jnp.log(_u01(bits, TINY, jnp.float32(1.0))))


def _normal(bits):
    return SQRT2 * jax.lax.erf_inv(_u01(bits, LO_N, jnp.float32(1.0)))


def _tree8(a):
    b = a[:, :4] + a[:, 4:]
    c = b[:, :2] + b[:, 2:]
    return c[:, 0:1] + c[:, 1:2]


def _red64(t):
    acc = t[:, 0:8]
    for v in range(1, 8):
        acc = acc + t[:, 8 * v:8 * v + 8]
    return _tree8(acc)


def _red16(t):
    return _tree8(t[:, 0:8] + t[:, 8:16])


def _prep_kernel(keys_ref, p_ref, krep_ref, q_ref, rrep_ref, t_ref, lwu_ref):
    k1 = keys_ref[0]
    k2 = keys_ref[1]
    P = p_ref[...]
    mean = jnp.dot(P, krep_ref[...], preferred_element_type=jnp.float32)
    r = jax.lax.broadcasted_iota(jnp.uint32, (NUM_P, 128), 0)
    l = jax.lax.broadcasted_iota(jnp.uint32, (NUM_P, 128), 1)
    eps = _normal(_bits(k1, k2, r * jnp.uint32(128) + l))
    nu = mean + jnp.float32(0.1) * eps
    t_ref[:, 0:DX] = P
    t_ref[:, DX:DX + 128] = nu
    t_ref[:, DX + 128:] = jnp.zeros((NUM_P, 256 - DX - 128), jnp.float32)
    costx = _red64(P * P * q_ref[...])
    t = nu * nu * rrep_ref[...]
    cols = []
    for u in range(8):
        cu = _red16(t[:, 16 * u:16 * u + 16])
        cols.append(costx + cu)
    lwu_ref[...] = -jnp.concatenate(cols, axis=1)


def _lse_kernel(lwu_ref, logw_ref):
    lwu = lwu_ref[...]
    m = jnp.max(lwu)
    s = jnp.sum(jnp.exp(lwu - m))
    lse = jnp.log(jnp.abs(s)) + m
    logw_ref[...] = lwu - lse


_NW = 32

_IB = 32
_UNROLL = 4


def _argmax_kernel(keys_ref, logw_ref, samp_ref, lwsel_ref):
    k1 = keys_ref[2]
    k2 = keys_ref[3]
    i0 = pl.program_id(0) * _IB
    s_iota = jax.lax.broadcasted_iota(jnp.uint32, (_IB, 128), 0)
    l_iota = jax.lax.broadcasted_iota(jnp.uint32, (_IB, 128), 1)
    pbase = (jnp.uint32(i0) + s_iota) * jnp.uint32(NR) + l_iota
    l_i32 = jax.lax.broadcasted_iota(jnp.int32, (_IB, 128), 1)

    neg_inf = jnp.full((_IB, 128), -jnp.inf, jnp.float32)
    zero_i = jnp.zeros((_IB, 128), jnp.int32)

    def body(c, carry):
        vb, jb, lb = carry
        for u in range(_UNROLL):
            row = c * _UNROLL + u
            j0 = row * 128
            lw = logw_ref[pl.ds(row, 1), :]
            lw = jnp.broadcast_to(lw, (_IB, 128))
            g = _gumbel(_bits(k1, k2, pbase + jnp.uint32(j0)))
            v = g + lw
            upd = v > vb
            vb = jnp.where(upd, v, vb)
            jb = jnp.where(upd, l_i32 + j0, jb)
            lb = jnp.where(upd, lw, lb)
        return vb, jb, lb

    vb, jb, lb = jax.lax.fori_loop(
        0, 256 // _UNROLL, body, (neg_inf, zero_i, neg_inf))

    vmax = jnp.max(vb, axis=1, keepdims=True)
    mask = vb == vmax
    jcand = jnp.where(mask, jb, jnp.int32(2147483647))
    jmin = jnp.min(jcand, axis=1, keepdims=True)
    sel = mask & (jb == jmin)
    lwin = jnp.min(jnp.where(sel, lb, jnp.inf), axis=1, keepdims=True)
    samp_ref[...] = jmin
    lwsel_ref[...] = lwin


_BPW = NUM_P // _NW


def _sc_gather_body(idx_hbm, t_hbm, g_hbm,
                    idx_v, idx2_v, g_v, sem):
    wid = lax.axis_index("s") * 2 + lax.axis_index("c")
    base = wid * _BPW
    pltpu.sync_copy(idx_hbm.at[pl.ds(base, _BPW)], idx_v)
    for i in range(_BPW // 16):
        sl = pl.ds(i * 16, 16)
        idx2_v[sl] = lax.shift_right_arithmetic(idx_v[sl], 3)
    pltpu.async_copy(t_hbm.at[idx2_v], g_v, sem).wait()
    pltpu.sync_copy(g_v, g_hbm.at[pl.ds(base, _BPW)])


def _sc_gather():
    return pl.kernel(
        _sc_gather_body,
        mesh=plsc.VectorSubcoreMesh(core_axis_name="c", subcore_axis_name="s"),
        out_type=jax.ShapeDtypeStruct((NUM_P, 256), jnp.float32),
        scratch_types=[pltpu.VMEM((_BPW,), jnp.int32),
                       pltpu.VMEM((_BPW,), jnp.int32),
                       pltpu.VMEM((_BPW, 256), jnp.float32),
                       pltpu.SemaphoreType.DMA])


def _dyn_kernel(keys_ref, g_ref, samp_ref, at_ref, bt_ref, out_ref, us_ref):
    k1 = keys_ref[4]
    k2 = keys_ref[5]
    xs = g_ref[:, 0:DX]
    rep = lax.rem(samp_ref[...], jnp.int32(US))
    us = jnp.zeros((NUM_P, DU), jnp.float32)
    for u in range(US):
        sl = g_ref[:, DX + 16 * u:DX + 16 * u + 16]
        us = jnp.where(rep == u, sl, us)
    us_ref[...] = us
    x1 = jnp.dot(xs, at_ref[...], preferred_element_type=jnp.float32)
    x2 = jnp.dot(us, bt_ref[...], preferred_element_type=jnp.float32)
    r = jax.lax.broadcasted_iota(jnp.uint32, (NUM_P, DX), 0)
    l = jax.lax.broadcasted_iota(jnp.uint32, (NUM_P, DX), 1)
    eps = _normal(_bits(k1, k2, r * jnp.uint32(DX) + l))
    out_ref[...] = x1 + x2 + jnp.float32(0.01) * eps


def kernel(particles, K, A, B, Q, R, iteration):
    key = jax.random.fold_in(jax.random.key(1234), iteration)
    ku, ks, kn = jax.random.split(key, 3)
    keys = jnp.concatenate([jax.random.key_data(ku),
                            jax.random.key_data(ks),
                            jax.random.key_data(kn)]).astype(jnp.uint32)

    krep = jnp.tile(K.T, (1, US))
    rrep = jnp.tile(R.reshape(1, DU), (1, US))

    table, lwu = pl.pallas_call(
        _prep_kernel,
        in_specs=[pl.BlockSpec(memory_space=pltpu.SMEM),
                  pl.BlockSpec((NUM_P, DX), lambda: (0, 0)),
                  pl.BlockSpec((DX, 128), lambda: (0, 0)),
                  pl.BlockSpec((1, DX), lambda: (0, 0)),
                  pl.BlockSpec((1, 128), lambda: (0, 0))],
        out_specs=[pl.BlockSpec((NUM_P, 256), lambda: (0, 0)),
                   pl.BlockSpec((NUM_P, US), lambda: (0, 0))],
        out_shape=[jax.ShapeDtypeStruct((NUM_P, 256), jnp.float32),
                   jax.ShapeDtypeStruct((NUM_P, US), jnp.float32)],
    )(keys, particles, krep, Q.reshape(1, DX), rrep)

    logw = pl.pallas_call(
        _lse_kernel,
        in_specs=[pl.BlockSpec((256, 128), lambda: (0, 0))],
        out_specs=pl.BlockSpec((256, 128), lambda: (0, 0)),
        out_shape=jax.ShapeDtypeStruct((256, 128), jnp.float32),
    )(lwu.reshape(256, 128))

    samp, lwsel = pl.pallas_call(
        _argmax_kernel,
        grid=(NUM_P // _IB,),
        in_specs=[pl.BlockSpec(memory_space=pltpu.SMEM),
                  pl.BlockSpec((256, 128), lambda i: (0, 0))],
        out_specs=[pl.BlockSpec((_IB, 1), lambda i: (i, 0)),
                   pl.BlockSpec((_IB, 1), lambda i: (i, 0))],
        out_shape=[jax.ShapeDtypeStruct((NUM_P, 1), jnp.int32),
                   jax.ShapeDtypeStruct((NUM_P, 1), jnp.float32)],
    )(keys, logw)

    samples = samp.reshape(NUM_P)

    g = _sc_gather()(samples, table)

    new_particles, gu = pl.pallas_call(
        _dyn_kernel,
        in_specs=[pl.BlockSpec(memory_space=pltpu.SMEM),
                  pl.BlockSpec((NUM_P, 256), lambda: (0, 0)),
                  pl.BlockSpec((NUM_P, 1), lambda: (0, 0)),
                  pl.BlockSpec((DX, DX), lambda: (0, 0)),
                  pl.BlockSpec((DU, DX), lambda: (0, 0))],
        out_specs=[pl.BlockSpec((NUM_P, DX), lambda: (0, 0)),
                   pl.BlockSpec((NUM_P, DU), lambda: (0, 0))],
        out_shape=[jax.ShapeDtypeStruct((NUM_P, DX), jnp.float32),
                   jax.ShapeDtypeStruct((NUM_P, DU), jnp.float32)],
    )(keys, g, samp, A.T, B.T)

    joint = jnp.concatenate([g[:, 0:DX], gu], axis=1)
    log_weights = lwsel.reshape(NUM_P)
    return (new_particles, joint, log_weights)
